# Initial kernel scaffold; baseline (speedup 1.0000x reference)
import jax
import jax.numpy as jnp
from jax.experimental import pallas as pl


def kernel(audio_ids, tables):
    raise NotImplementedError("write your pallas kernel here")


# speedup vs baseline: 2.4898x; 2.4898x over previous
import functools

import jax
import jax.numpy as jnp
from jax import lax
from jax.experimental import pallas as pl
from jax.experimental.pallas import tpu as pltpu
from jax.experimental.pallas import tpu_sc as plsc

C = 8
V = 1024
H = 2048
B = 4
S = 2048
SCALE = 10.0

NW = 32
TOKENS = S * B
T_PER_W = TOKENS // NW
K = 2
NCHUNK = T_PER_W // K
ROWS_PER_CHUNK = K * C


def _body(table_hbm, idx_hbm, out_hbm,
          idx_v, gbuf0, gbuf1, obuf0, obuf1,
          gsem0, gsem1, osem0, osem1):
  nc = 2
  wid = lax.axis_index("s") * nc + lax.axis_index("c")
  row0 = wid * NCHUNK
  tok0 = wid * T_PER_W

  pltpu.sync_copy(idx_hbm.at[pl.ds(row0, NCHUNK)], idx_v)
  offs = (lax.iota(jnp.int32, 16) & 7) * V

  @pl.loop(0, NCHUNK)
  def _(r):
    idx_v[r] = idx_v[r] + offs

  gbufs = (gbuf0, gbuf1)
  gsems = (gsem0, gsem1)
  obufs = (obuf0, obuf1)
  osems = (osem0, osem1)

  def start_gather(chunk, b):
    pltpu.async_copy(table_hbm.at[idx_v.at[chunk]], gbufs[b], gsems[b])

  def wait_gather(b):
    pltpu.make_async_copy(table_hbm.at[idx_v.at[0]], gbufs[b], gsems[b]).wait()

  start_gather(0, 0)
  start_gather(1, 1)

  @pl.loop(0, NCHUNK, step=2)
  def _(g):
    for b in range(2):
      gc = g + b
      wait_gather(b)
      @pl.when(gc >= 2)
      def _():
        pltpu.make_async_copy(
            obufs[b], out_hbm.at[pl.ds(tok0, K)], osems[b]).wait()

      gbuf = gbufs[b]
      obuf = obufs[b]
      for k in range(K):
        @pl.loop(0, H, step=16)
        def _(j):
          col = pl.ds(j, 16)
          acc = gbuf[k * C, col]
          for c in range(1, C):
            acc = acc + gbuf[k * C + c, col]
          obuf[k, col] = acc * SCALE

      pltpu.async_copy(obufs[b], out_hbm.at[pl.ds(tok0 + gc * K, K)],
                       osems[b])

      @pl.when(gc + 2 < NCHUNK)
      def _():
        start_gather(gc + 2, b)

  for b in range(2):
    pltpu.make_async_copy(obufs[b], out_hbm.at[pl.ds(tok0, K)],
                          osems[b]).wait()


@jax.jit
def _run(table_flat, idx2d):
  mesh = plsc.VectorSubcoreMesh(core_axis_name="c", subcore_axis_name="s")
  return pl.kernel(
      _body,
      out_type=jax.ShapeDtypeStruct((TOKENS, H), jnp.float32),
      mesh=mesh,
      scratch_types=[
          pltpu.VMEM((NCHUNK, 16), jnp.int32),
          pltpu.VMEM((ROWS_PER_CHUNK, H), jnp.float32),
          pltpu.VMEM((ROWS_PER_CHUNK, H), jnp.float32),
          pltpu.VMEM((K, H), jnp.float32),
          pltpu.VMEM((K, H), jnp.float32),
          pltpu.SemaphoreType.DMA,
          pltpu.SemaphoreType.DMA,
          pltpu.SemaphoreType.DMA,
          pltpu.SemaphoreType.DMA,
      ],
  )(table_flat, idx2d)


def kernel(audio_ids, tables):
  ids = jnp.transpose(audio_ids, (1, 0, 2))
  idx2d = ids.reshape(TOKENS * C // 16, 16)
  table_flat = tables.reshape(C * V, H)
  out = _run(table_flat, idx2d)
  return out.reshape(S, B, H)
